# baseline (speedup 1.0000x reference)
import functools

import jax
import jax.numpy as jnp
from jax import lax
from jax.experimental import pallas as pl
from jax.experimental.pallas import tpu as pltpu
from jax.experimental.pallas import tpu_sc as plsc

N = 10000
E = 320000
D = 128
EPS = 1e-5

NC = 2
NS = 16
NW = NC * NS
CH = 128
G = 2
NG = 40
EPW = G * CH * NG
EPAD = NW * EPW
NDUMMY = 64
NA = N + NDUMMY
RPT = 624
TAIL = N - NS * RPT


def _segsum_body(table, src, dst, out,
                 isrc, idst, rows, acc, *sems):
    c = lax.axis_index("c")
    s = lax.axis_index("s")
    wid = s * NC + c
    gsems = sems[:G]
    ssems = sems[G:2 * G]
    isem = sems[2 * G]

    def zrow(i, _):
        for j in range(D // 16):
            rows[0, i, pl.ds(j * 16, 16)] = jnp.zeros((16,), jnp.float32)
        return 0

    lax.fori_loop(0, CH, zrow, 0)
    for k in range(RPT // CH):
        pltpu.sync_copy(rows.at[0], acc.at[pl.ds(s * RPT + k * CH, CH)])
    rem = RPT - (RPT // CH) * CH
    if rem:
        pltpu.sync_copy(rows.at[0, pl.ds(0, rem)],
                        acc.at[pl.ds(s * RPT + (RPT // CH) * CH, rem)])

    @pl.when(s == NS - 1)
    def _():
        pltpu.sync_copy(rows.at[0, pl.ds(0, TAIL)],
                        acc.at[pl.ds(NS * RPT, TAIL)])

    plsc.subcore_barrier()

    pltpu.sync_copy(src.at[wid, 0], isrc.at[0])
    pltpu.sync_copy(dst.at[wid, 0], idst.at[0])
    for b in range(G):
        pltpu.async_copy(table.at[isrc.at[0, b]], rows.at[b], gsems[b])

    def group(g, _):
        p = lax.rem(g, 2)
        p1 = lax.rem(g + 1, 2)
        more = g < NG - 1

        @pl.when(more)
        def _():
            pltpu.async_copy(src.at[wid, g + 1], isrc.at[p1], isem)
            pltpu.async_copy(dst.at[wid, g + 1], idst.at[p1], isem)

        for b in range(G):
            pltpu.make_async_copy(
                table.at[isrc.at[p, b]], rows.at[b], gsems[b]).wait()
            pltpu.async_copy(rows.at[b], acc.at[idst.at[p, b]], ssems[b],
                             add=True)

        @pl.when(more)
        def _():
            pltpu.make_async_copy(src.at[wid, g + 1], isrc.at[p1], isem).wait()
            pltpu.make_async_copy(dst.at[wid, g + 1], idst.at[p1], isem).wait()

        for b in range(G):
            pltpu.make_async_copy(
                rows.at[b], acc.at[idst.at[p, b]], ssems[b]).wait()

            @pl.when(more)
            def _():
                pltpu.async_copy(
                    table.at[isrc.at[p1, b]], rows.at[b], gsems[b])

        return 0

    lax.fori_loop(0, NG, group, 0)
    plsc.subcore_barrier()

    pltpu.sync_copy(acc.at[pl.ds(s * RPT, RPT)], out.at[c, pl.ds(s * RPT, RPT)])

    @pl.when(s == NS - 1)
    def _():
        pltpu.sync_copy(acc.at[pl.ds(NS * RPT, TAIL)],
                        out.at[c, pl.ds(NS * RPT, TAIL)])


_segsum = functools.partial(
    pl.kernel,
    out_type=jax.ShapeDtypeStruct((NC, N, D), jnp.float32),
    mesh=plsc.VectorSubcoreMesh(core_axis_name="c", subcore_axis_name="s"),
    scratch_types=[
        pltpu.VMEM((2, G, CH), jnp.int32),
        pltpu.VMEM((2, G, CH), jnp.int32),
        pltpu.VMEM((G, CH, D), jnp.float32),
        pltpu.VMEM_SHARED((NA, D), jnp.float32),
    ] + [pltpu.SemaphoreType.DMA] * (2 * G + 1),
)(_segsum_body)


def _tc_body(p_ref, w_ref, s_ref, b_ref, res_ref, out_ref, *, relu, resid):
    agg = p_ref[0] + p_ref[1]
    y = jnp.dot(agg, w_ref[...], preferred_element_type=jnp.float32)
    mean = jnp.mean(y, axis=0, keepdims=True)
    cen = y - mean
    var = jnp.mean(cen * cen, axis=0, keepdims=True)
    z = cen * lax.rsqrt(var + EPS) * s_ref[...] + b_ref[...]
    if relu:
        z = jnp.maximum(z, 0.0)
    if resid:
        z = z + res_ref[...]
    out_ref[...] = z


def _tc_stage(p, w, s, b, res, *, relu):
    resid = res is not None
    body = functools.partial(_tc_body, relu=relu, resid=resid)
    if not resid:
        def fn(p_ref, w_ref, s_ref, b_ref, out_ref):
            body(p_ref, w_ref, s_ref, b_ref, None, out_ref)
        args = (p, w, s.reshape(1, D), b.reshape(1, D))
    else:
        fn = body
        args = (p, w, s.reshape(1, D), b.reshape(1, D), res)
    return pl.pallas_call(
        fn,
        out_shape=jax.ShapeDtypeStruct((N, D), jnp.float32),
    )(*args)


def kernel(x, edge_index, W1a, s1a, b1a, W2a, s2a, b2a,
           W1b, s1b, b1b, W2b, s2b, b2b):
    npad = EPAD - E
    pad_iota = jnp.arange(npad, dtype=jnp.int32)
    src = jnp.concatenate([edge_index[0], pad_iota % N]).reshape(NW, NG, G, CH)
    dst = jnp.concatenate([edge_index[1], N + pad_iota % NDUMMY]
                          ).reshape(NW, NG, G, CH)

    p = _segsum(x, src, dst)
    h = _tc_stage(p, W1a, s1a, b1a, None, relu=True)
    p = _segsum(h, src, dst)
    x1 = _tc_stage(p, W2a, s2a, b2a, x, relu=False)
    p = _segsum(x1, src, dst)
    h = _tc_stage(p, W1b, s1b, b1b, None, relu=True)
    p = _segsum(h, src, dst)
    out = _tc_stage(p, W2b, s2b, b2b, x1, relu=False)
    return out


# speedup vs baseline: 1.2128x; 1.2128x over previous
import functools

import jax
import jax.numpy as jnp
from jax import lax
from jax.experimental import pallas as pl
from jax.experimental.pallas import tpu as pltpu
from jax.experimental.pallas import tpu_sc as plsc

N = 10000
E = 320000
D = 128
EPS = 1e-5

NC = 2
NS = 16
NW = NC * NS
CH = 40
G = 8
NG = 32
EPW = G * CH * NG
EPAD = NW * EPW
NDUMMY = 64
NA = N + NDUMMY
RPT = 624
TAIL = N - NS * RPT


def _segsum_body(table, src, dst, out,
                 isrc, idst, rows, acc, *sems):
    c = lax.axis_index("c")
    s = lax.axis_index("s")
    wid = s * NC + c
    gsems = sems[:G]
    ssems = sems[G:2 * G]
    isem = sems[2 * G]

    def zrow(i, _):
        for j in range(D // 16):
            rows[0, i, pl.ds(j * 16, 16)] = jnp.zeros((16,), jnp.float32)
        return 0

    lax.fori_loop(0, CH, zrow, 0)
    for k in range(RPT // CH):
        pltpu.sync_copy(rows.at[0], acc.at[pl.ds(s * RPT + k * CH, CH)])
    rem = RPT - (RPT // CH) * CH
    if rem:
        pltpu.sync_copy(rows.at[0, pl.ds(0, rem)],
                        acc.at[pl.ds(s * RPT + (RPT // CH) * CH, rem)])

    @pl.when(s == NS - 1)
    def _():
        pltpu.sync_copy(rows.at[0, pl.ds(0, TAIL)],
                        acc.at[pl.ds(NS * RPT, TAIL)])

    plsc.subcore_barrier()

    pltpu.sync_copy(src.at[wid, 0], isrc.at[0])
    pltpu.sync_copy(dst.at[wid, 0], idst.at[0])
    for b in range(G):
        pltpu.async_copy(table.at[isrc.at[0, b]], rows.at[b], gsems[b])

    def group(g, _):
        p = lax.rem(g, 2)
        p1 = lax.rem(g + 1, 2)
        more = g < NG - 1

        @pl.when(more)
        def _():
            pltpu.async_copy(src.at[wid, g + 1], isrc.at[p1], isem)
            pltpu.async_copy(dst.at[wid, g + 1], idst.at[p1], isem)

        for b in range(G):
            pltpu.make_async_copy(
                table.at[isrc.at[p, b]], rows.at[b], gsems[b]).wait()
            pltpu.async_copy(rows.at[b], acc.at[idst.at[p, b]], ssems[b],
                             add=True)

        @pl.when(more)
        def _():
            pltpu.make_async_copy(src.at[wid, g + 1], isrc.at[p1], isem).wait()
            pltpu.make_async_copy(dst.at[wid, g + 1], idst.at[p1], isem).wait()

        for b in range(G):
            pltpu.make_async_copy(
                rows.at[b], acc.at[idst.at[p, b]], ssems[b]).wait()

            @pl.when(more)
            def _():
                pltpu.async_copy(
                    table.at[isrc.at[p1, b]], rows.at[b], gsems[b])

        return 0

    lax.fori_loop(0, NG, group, 0)
    plsc.subcore_barrier()

    pltpu.sync_copy(acc.at[pl.ds(s * RPT, RPT)], out.at[c, pl.ds(s * RPT, RPT)])

    @pl.when(s == NS - 1)
    def _():
        pltpu.sync_copy(acc.at[pl.ds(NS * RPT, TAIL)],
                        out.at[c, pl.ds(NS * RPT, TAIL)])


_segsum = functools.partial(
    pl.kernel,
    out_type=jax.ShapeDtypeStruct((NC, N, D), jnp.float32),
    mesh=plsc.VectorSubcoreMesh(core_axis_name="c", subcore_axis_name="s"),
    scratch_types=[
        pltpu.VMEM((2, G, CH), jnp.int32),
        pltpu.VMEM((2, G, CH), jnp.int32),
        pltpu.VMEM((G, CH, D), jnp.float32),
        pltpu.VMEM_SHARED((NA, D), jnp.float32),
    ] + [pltpu.SemaphoreType.DMA] * (2 * G + 1),
)(_segsum_body)


def _tc_body(p_ref, w_ref, s_ref, b_ref, res_ref, out_ref, *, relu, resid):
    agg = p_ref[0] + p_ref[1]
    y = jnp.dot(agg, w_ref[...], preferred_element_type=jnp.float32)
    mean = jnp.mean(y, axis=0, keepdims=True)
    cen = y - mean
    var = jnp.mean(cen * cen, axis=0, keepdims=True)
    z = cen * lax.rsqrt(var + EPS) * s_ref[...] + b_ref[...]
    if relu:
        z = jnp.maximum(z, 0.0)
    if resid:
        z = z + res_ref[...]
    out_ref[...] = z


def _tc_stage(p, w, s, b, res, *, relu):
    resid = res is not None
    body = functools.partial(_tc_body, relu=relu, resid=resid)
    if not resid:
        def fn(p_ref, w_ref, s_ref, b_ref, out_ref):
            body(p_ref, w_ref, s_ref, b_ref, None, out_ref)
        args = (p, w, s.reshape(1, D), b.reshape(1, D))
    else:
        fn = body
        args = (p, w, s.reshape(1, D), b.reshape(1, D), res)
    return pl.pallas_call(
        fn,
        out_shape=jax.ShapeDtypeStruct((N, D), jnp.float32),
    )(*args)


def kernel(x, edge_index, W1a, s1a, b1a, W2a, s2a, b2a,
           W1b, s1b, b1b, W2b, s2b, b2b):
    npad = EPAD - E
    pad_iota = jnp.arange(npad, dtype=jnp.int32)
    src = jnp.concatenate([edge_index[0], pad_iota % N]).reshape(NW, NG, G, CH)
    dst = jnp.concatenate([edge_index[1], N + pad_iota % NDUMMY]
                          ).reshape(NW, NG, G, CH)

    p = _segsum(x, src, dst)
    h = _tc_stage(p, W1a, s1a, b1a, None, relu=True)
    p = _segsum(h, src, dst)
    x1 = _tc_stage(p, W2a, s2a, b2a, x, relu=False)
    p = _segsum(x1, src, dst)
    h = _tc_stage(p, W1b, s1b, b1b, None, relu=True)
    p = _segsum(h, src, dst)
    out = _tc_stage(p, W2b, s2b, b2b, x1, relu=False)
    return out


# speedup vs baseline: 1.2450x; 1.0265x over previous
import functools

import jax
import jax.numpy as jnp
from jax import lax
from jax.experimental import pallas as pl
from jax.experimental.pallas import tpu as pltpu
from jax.experimental.pallas import tpu_sc as plsc

N = 10000
E = 320000
D = 128
EPS = 1e-5

NC = 2
NS = 16
NW = NC * NS
CH = 64
G = 5
NG = 32
EPW = G * CH * NG
EPAD = NW * EPW
NDUMMY = 64
NA = N + NDUMMY
RPT = 624
TAIL = N - NS * RPT


def _segsum_body(table, src, dst, out,
                 isrc, idst, rows, acc, *sems):
    c = lax.axis_index("c")
    s = lax.axis_index("s")
    wid = s * NC + c
    gsems = sems[:G]
    ssems = sems[G:2 * G]
    isem = sems[2 * G]

    def zrow(i, _):
        for j in range(D // 16):
            rows[0, i, pl.ds(j * 16, 16)] = jnp.zeros((16,), jnp.float32)
        return 0

    lax.fori_loop(0, CH, zrow, 0)
    for k in range(RPT // CH):
        pltpu.sync_copy(rows.at[0], acc.at[pl.ds(s * RPT + k * CH, CH)])
    rem = RPT - (RPT // CH) * CH
    if rem:
        pltpu.sync_copy(rows.at[0, pl.ds(0, rem)],
                        acc.at[pl.ds(s * RPT + (RPT // CH) * CH, rem)])

    @pl.when(s == NS - 1)
    def _():
        pltpu.sync_copy(rows.at[0, pl.ds(0, TAIL)],
                        acc.at[pl.ds(NS * RPT, TAIL)])

    plsc.subcore_barrier()

    pltpu.sync_copy(src.at[wid, 0], isrc.at[0])
    pltpu.sync_copy(dst.at[wid, 0], idst.at[0])
    for b in range(G):
        pltpu.async_copy(table.at[isrc.at[0, b]], rows.at[b], gsems[b])

    def group(g, _):
        p = lax.rem(g, 2)
        p1 = lax.rem(g + 1, 2)
        more = g < NG - 1

        @pl.when(more)
        def _():
            pltpu.async_copy(src.at[wid, g + 1], isrc.at[p1], isem)
            pltpu.async_copy(dst.at[wid, g + 1], idst.at[p1], isem)

        for b in range(G):
            pltpu.make_async_copy(
                table.at[isrc.at[p, b]], rows.at[b], gsems[b]).wait()
            pltpu.async_copy(rows.at[b], acc.at[idst.at[p, b]], ssems[b],
                             add=True)

        @pl.when(more)
        def _():
            pltpu.make_async_copy(src.at[wid, g + 1], isrc.at[p1], isem).wait()
            pltpu.make_async_copy(dst.at[wid, g + 1], idst.at[p1], isem).wait()

        for b in range(G):
            pltpu.make_async_copy(
                rows.at[b], acc.at[idst.at[p, b]], ssems[b]).wait()

            @pl.when(more)
            def _():
                pltpu.async_copy(
                    table.at[isrc.at[p1, b]], rows.at[b], gsems[b])

        return 0

    lax.fori_loop(0, NG, group, 0)
    plsc.subcore_barrier()

    pltpu.sync_copy(acc.at[pl.ds(s * RPT, RPT)], out.at[c, pl.ds(s * RPT, RPT)])

    @pl.when(s == NS - 1)
    def _():
        pltpu.sync_copy(acc.at[pl.ds(NS * RPT, TAIL)],
                        out.at[c, pl.ds(NS * RPT, TAIL)])


_segsum = functools.partial(
    pl.kernel,
    out_type=jax.ShapeDtypeStruct((NC, N, D), jnp.float32),
    mesh=plsc.VectorSubcoreMesh(core_axis_name="c", subcore_axis_name="s"),
    scratch_types=[
        pltpu.VMEM((2, G, CH), jnp.int32),
        pltpu.VMEM((2, G, CH), jnp.int32),
        pltpu.VMEM((G, CH, D), jnp.float32),
        pltpu.VMEM_SHARED((NA, D), jnp.float32),
    ] + [pltpu.SemaphoreType.DMA] * (2 * G + 1),
)(_segsum_body)


def _tc_body(p_ref, w_ref, s_ref, b_ref, res_ref, out_ref, *, relu, resid):
    agg = p_ref[0] + p_ref[1]
    y = jnp.dot(agg, w_ref[...], preferred_element_type=jnp.float32)
    mean = jnp.mean(y, axis=0, keepdims=True)
    cen = y - mean
    var = jnp.mean(cen * cen, axis=0, keepdims=True)
    z = cen * lax.rsqrt(var + EPS) * s_ref[...] + b_ref[...]
    if relu:
        z = jnp.maximum(z, 0.0)
    if resid:
        z = z + res_ref[...]
    out_ref[...] = z


def _tc_stage(p, w, s, b, res, *, relu):
    resid = res is not None
    body = functools.partial(_tc_body, relu=relu, resid=resid)
    if not resid:
        def fn(p_ref, w_ref, s_ref, b_ref, out_ref):
            body(p_ref, w_ref, s_ref, b_ref, None, out_ref)
        args = (p, w, s.reshape(1, D), b.reshape(1, D))
    else:
        fn = body
        args = (p, w, s.reshape(1, D), b.reshape(1, D), res)
    return pl.pallas_call(
        fn,
        out_shape=jax.ShapeDtypeStruct((N, D), jnp.float32),
    )(*args)


def kernel(x, edge_index, W1a, s1a, b1a, W2a, s2a, b2a,
           W1b, s1b, b1b, W2b, s2b, b2b):
    npad = EPAD - E
    pad_iota = jnp.arange(npad, dtype=jnp.int32)
    src = jnp.concatenate([edge_index[0], pad_iota % N]).reshape(NW, NG, G, CH)
    dst = jnp.concatenate([edge_index[1], N + pad_iota % NDUMMY]
                          ).reshape(NW, NG, G, CH)

    p = _segsum(x, src, dst)
    h = _tc_stage(p, W1a, s1a, b1a, None, relu=True)
    p = _segsum(h, src, dst)
    x1 = _tc_stage(p, W2a, s2a, b2a, x, relu=False)
    p = _segsum(x1, src, dst)
    h = _tc_stage(p, W1b, s1b, b1b, None, relu=True)
    p = _segsum(h, src, dst)
    out = _tc_stage(p, W2b, s2b, b2b, x1, relu=False)
    return out
